# baseline (speedup 1.0000x reference)
import functools

import jax
import jax.numpy as jnp
from jax import lax
from jax.experimental import pallas as pl
from jax.experimental.pallas import tpu as pltpu
from jax.experimental.pallas import tpu_sc as plsc

_B, _S = 1024, 200
_D = 32
_V = 1000000
_L = 128
_NBUF = 2


@functools.cache
def _build_gather():
    info = plsc.get_sparse_core_info()
    nw = info.num_cores * info.num_subcores
    n_ti = _B // _L
    n_jgrp = nw // n_ti
    j_per_w = _S // n_jgrp
    tj_span = j_per_w // 8 + 1

    mesh = plsc.VectorSubcoreMesh(core_axis_name="c", subcore_axis_name="s")

    @functools.partial(
        pl.kernel,
        out_type=jax.ShapeDtypeStruct((_S, _D // 8, n_ti, 8, _L), jnp.float32),
        mesh=mesh,
        scratch_types=[
            pltpu.VMEM((tj_span, 8, _L), jnp.int32),
            pltpu.VMEM((_NBUF, 32, 8, _D), jnp.float32),
            pltpu.VMEM((_NBUF, _D // 8, 8, _L), jnp.float32),
            pltpu.SemaphoreType.DMA((_NBUF,)),
            pltpu.SemaphoreType.DMA((_NBUF,)),
            pltpu.SemaphoreType.DMA,
        ],
        compiler_params=pltpu.CompilerParams(
            use_tc_tiling_on_sc=True, needs_layout_passes=False
        ),
    )
    def gather(tid_hbm, table_hbm, out_hbm, idx_v, gbuf, obuf,
               gsems, osems, isem):
        wid = lax.axis_index("s") * info.num_cores + lax.axis_index("c")
        ti = wid % n_ti
        j0 = (wid // n_ti) * j_per_w
        tj0 = j0 // 8

        pltpu.async_copy(
            tid_hbm.at[pl.ds(tj0, tj_span), ti], idx_v, isem
        ).wait()

        iota = lax.iota(jnp.int32, 16)
        n_sub = j_per_w * 4

        def start_gather(u, sub, b):
            j = j0 + u
            tj = j // 8 - tj0
            j8 = j % 8
            for g in range(2):
                iv = idx_v[tj, j8, pl.ds(sub * 32 + g * 16, 16)]
                tv = lax.shift_right_logical(iv, 3)
                for k in range(16):
                    pltpu.async_copy(table_hbm.at[tv[k]],
                                     gbuf.at[b, g * 16 + k], gsems.at[b])

        start_gather(0, 0, 0)
        start_gather(0, 1, 1)

        def body(n, carry):
            for ub in range(2):
                u = n * 2 + ub
                j = j0 + u
                tj = j // 8 - tj0
                j8 = j % 8
                for sub in range(4):
                    b = sub % 2
                    pltpu.make_async_copy(
                        table_hbm.at[pl.ds(0, 32)], gbuf.at[b], gsems.at[b]
                    ).wait()
                    if sub == 0:
                        @pl.when(u >= _NBUF)
                        def _drain():
                            pltpu.make_async_copy(
                                obuf.at[ub], out_hbm.at[0, :, 0],
                                osems.at[ub]
                            ).wait()

                    base = sub * 32
                    ivs = [idx_v[tj, j8, pl.ds(base + g * 16, 16)]
                           for g in range(2)]
                    r8s = [lax.bitwise_and(iv, 7) for iv in ivs]
                    rows = [iota, iota + 16]
                    poss = [iota + base, iota + (base + 16)]
                    def xpose(ccq, carry):
                        for ccs in range(4):
                            cc = ccq * 4 + ccs
                            cl = (lax.bitwise_and(iota + cc, 15)
                                  + lax.bitwise_and(cc, 16))
                            c1 = lax.shift_right_logical(cl, 3)
                            c2 = lax.bitwise_and(cl, 7)
                            for g in range(2):
                                vals = plsc.load_gather(
                                    gbuf.at[b], [rows[g], r8s[g], cl]
                                )
                                plsc.store_scatter(
                                    obuf.at[ub], [c1, c2, poss[g]], vals
                                )
                        return carry

                    lax.fori_loop(0, _D // 4, xpose, None)
                    if sub == 3:
                        pltpu.async_copy(obuf.at[ub], out_hbm.at[j, :, ti],
                                         osems.at[ub])
                    s2 = u * 4 + sub + 2

                    @pl.when(s2 < n_sub)
                    def _refill():
                        start_gather(u + (sub + 2) // 4, (sub + 2) % 4, b)

            return carry

        lax.fori_loop(0, j_per_w // _NBUF, body, None)

        for b in range(_NBUF):
            pltpu.make_async_copy(
                obuf.at[b], out_hbm.at[0, :, 0], osems.at[b]
            ).wait()

    return gather


def kernel(token_ids, weight):
    tid4 = token_ids.T.reshape(_S // 8, 8, _B // _L, _L).transpose(0, 2, 1, 3)
    table = weight.reshape(_V // 8, 8, _D)
    out5 = _build_gather()(tid4, table)
    return out5.transpose(2, 4, 0, 1, 3).reshape(_B, _S, _D)


# speedup vs baseline: 1.0225x; 1.0225x over previous
import functools

import jax
import jax.numpy as jnp
from jax import lax
from jax.experimental import pallas as pl
from jax.experimental.pallas import tpu as pltpu
from jax.experimental.pallas import tpu_sc as plsc

_B, _S = 1024, 200
_D = 32
_V = 1000000
_L = 128
_NBUF = 2


@functools.cache
def _build_gather():
    info = plsc.get_sparse_core_info()
    nw = info.num_cores * info.num_subcores
    n_ti = _B // _L
    n_jgrp = nw // n_ti
    j_per_w = _S // n_jgrp
    tj_span = j_per_w // 8 + 1

    mesh = plsc.VectorSubcoreMesh(core_axis_name="c", subcore_axis_name="s")

    @functools.partial(
        pl.kernel,
        out_type=jax.ShapeDtypeStruct((_S, _D // 8, n_ti, 8, _L), jnp.float32),
        mesh=mesh,
        scratch_types=[
            pltpu.VMEM((tj_span, 8, _L), jnp.int32),
            pltpu.VMEM((_NBUF, 32, 8, _D), jnp.float32),
            pltpu.VMEM((_NBUF, _D // 8, 8, _L), jnp.float32),
            pltpu.SemaphoreType.DMA((_NBUF,)),
            pltpu.SemaphoreType.DMA((_NBUF,)),
            pltpu.SemaphoreType.DMA,
        ],
        compiler_params=pltpu.CompilerParams(
            use_tc_tiling_on_sc=True, needs_layout_passes=False
        ),
    )
    def gather(tid_hbm, table_hbm, out_hbm, idx_v, gbuf, obuf,
               gsems, osems, isem):
        wid = lax.axis_index("s") * info.num_cores + lax.axis_index("c")
        ti = wid % n_ti
        j0 = (wid // n_ti) * j_per_w
        tj0 = j0 // 8

        pltpu.async_copy(
            tid_hbm.at[pl.ds(tj0, tj_span), ti], idx_v, isem
        ).wait()

        iota = lax.iota(jnp.int32, 16)
        n_sub = j_per_w * 4

        def start_gather(u, sub, b):
            j = j0 + u
            tj = j // 8 - tj0
            j8 = j % 8
            for g in range(2):
                iv = idx_v[tj, j8, pl.ds(sub * 32 + g * 16, 16)]
                tv = lax.shift_right_logical(iv, 3)
                for k in range(16):
                    pltpu.async_copy(table_hbm.at[tv[k]],
                                     gbuf.at[b, g * 16 + k], gsems.at[b])

        start_gather(0, 0, 0)
        start_gather(0, 1, 1)

        def body(n, carry):
            for ub in range(2):
                u = n * 2 + ub
                j = j0 + u
                tj = j // 8 - tj0
                j8 = j % 8
                for sub in range(4):
                    b = sub % 2
                    pltpu.make_async_copy(
                        table_hbm.at[pl.ds(0, 32)], gbuf.at[b], gsems.at[b]
                    ).wait()
                    if sub == 0:
                        @pl.when(u >= _NBUF)
                        def _drain():
                            pltpu.make_async_copy(
                                obuf.at[ub], out_hbm.at[0, :, 0],
                                osems.at[ub]
                            ).wait()

                    for g in range(2):
                        iv = idx_v[tj, j8, pl.ds(sub * 32 + g * 16, 16)]
                        r8 = lax.bitwise_and(iv, 7)
                        rowv = iota + (g * 16)
                        for c in range(_D):
                            vals = plsc.load_gather(
                                gbuf.at[b],
                                [rowv, r8, jnp.full((16,), c, jnp.int32)],
                            )
                            obuf[ub, c // 8, c % 8,
                                 pl.ds(sub * 32 + g * 16, 16)] = vals
                    if sub == 3:
                        pltpu.async_copy(obuf.at[ub], out_hbm.at[j, :, ti],
                                         osems.at[ub])
                    s2 = u * 4 + sub + 2

                    @pl.when(s2 < n_sub)
                    def _refill():
                        start_gather(u + (sub + 2) // 4, (sub + 2) % 4, b)

            return carry

        lax.fori_loop(0, j_per_w // _NBUF, body, None)

        for b in range(_NBUF):
            pltpu.make_async_copy(
                obuf.at[b], out_hbm.at[0, :, 0], osems.at[b]
            ).wait()

    return gather


def kernel(token_ids, weight):
    tid4 = token_ids.T.reshape(_S // 8, 8, _B // _L, _L).transpose(0, 2, 1, 3)
    table = weight.reshape(_V // 8, 8, _D)
    out5 = _build_gather()(tid4, table)
    return out5.transpose(2, 4, 0, 1, 3).reshape(_B, _S, _D)
